# baseline (speedup 1.0000x reference)
import functools
import jax
import jax.numpy as jnp
from jax import lax
from jax.experimental import pallas as pl
from jax.experimental.pallas import tpu as pltpu

B = 1024
N = 100000
D = 64
K = 20
CHUNK = 2048
BUCKET = 128
NCHUNK = (N + CHUNK - 1) // CHUNK
NPAD = NCHUNK * CHUNK
NBUCKETS = NPAD // BUCKET
BPB = CHUNK // BUCKET
BT = 128
INF = float("inf")


def _d2_kernel(x_ref, data_ref, d2_ref, mins_ref):
    j = pl.program_id(0)
    x = x_ref[...]
    x2 = jnp.sum(x * x, axis=1, keepdims=True)
    mins = []
    for bsub in range(BPB):
        dat = data_ref[pl.ds(bsub * BUCKET, BUCKET), :]
        y2 = jnp.sum(dat * dat, axis=1)[None, :]
        xy = jax.lax.dot_general(x, dat, (((1,), (1,)), ((), ())),
                                 preferred_element_type=jnp.float32)
        d2 = x2 + y2 - 2.0 * xy
        cols = (j * CHUNK + bsub * BUCKET
                + jax.lax.broadcasted_iota(jnp.int32, (1, BUCKET), 1))
        d2 = jnp.where(cols < N, d2, INF)
        d2_ref[pl.ds(bsub * B, B), :] = d2
        mins.append(jnp.min(d2, axis=-1, keepdims=True))
    mins_ref[...] = jnp.concatenate(mins, axis=1)[None]


def _select_kernel(mins_ref, ids_ref):
    m = mins_ref[...]
    col = jax.lax.broadcasted_iota(jnp.int32, (BT, NBUCKETS), 1)
    ids = []
    for _ in range(K):
        cur = jnp.min(m, axis=1, keepdims=True)
        idx = jnp.min(jnp.where(m == cur, col, jnp.int32(2 ** 30)),
                      axis=1, keepdims=True)
        ids.append(idx)
        m = jnp.where(col == idx, INF, m)
    pad = jnp.full((BT, 1), NBUCKETS - 1, jnp.int32)
    ids.extend([pad] * (128 - K))
    ids_ref[...] = jnp.concatenate(ids, axis=1)


def _combine_kernel(z_ref, w_ref, b_ref, t_ref, u_ref, out_ref):
    z = z_ref[...]
    x = z[:, :D]
    w = w_ref[...]
    t = t_ref[0, 0]
    u = u_ref[...][:, :D]
    xdot = jax.lax.dot_general(x, w[:D, :], (((1,), (0,)), ((), ())),
                               preferred_element_type=jnp.float32)
    xdot = xdot + t * w[D:D + 1, :] + b_ref[...]
    num = jnp.sum(u * xdot, axis=1, keepdims=True)
    nu = jnp.sqrt(jnp.sum(u * u, axis=1, keepdims=True))
    nx = jnp.sqrt(jnp.sum(xdot * xdot, axis=1, keepdims=True))
    den = jnp.maximum(nu, 1e-8) * jnp.maximum(nx, 1e-8)
    cos = 1.0 - num / den
    l2 = jnp.sum((u - xdot) ** 2, axis=1, keepdims=True)
    out_ref[...] = jnp.concatenate([xdot, cos, l2], axis=1)


NW = 32
ROWS = B // NW
NSEL = 32
L = 16


def _sel(p, a, b):
    return (jnp.where(p, a[0], b[0]), jnp.where(p, a[1], b[1]))


def _rev(a):
    return (lax.rev(a[0], (0,)), lax.rev(a[1], (0,)))


def _sort1(a):
    from jax.experimental.pallas import tpu_sc as plsc
    k, v = plsc.sort_key_val(a[0], a[1])
    return (k, v)


def _merge16_16(a, b):
    rb = _rev(b)
    p = a[0] <= rb[0]
    lo = _sel(p, a, rb)
    hi = _sel(p, rb, a)
    return [_sort1(lo), _sort1(hi)]


def _merge32_low(A, Bb):
    rb1 = _rev(Bb[1])
    rb0 = _rev(Bb[0])
    p0 = A[0][0] <= rb1[0]
    l0 = _sel(p0, A[0], rb1)
    p1 = A[1][0] <= rb0[0]
    l1 = _sel(p1, A[1], rb0)
    q = l0[0] <= l1[0]
    m0 = _sel(q, l0, l1)
    m1 = _sel(q, l1, l0)
    return [_sort1(m0), _sort1(m1)]


def _tournament(blocks):
    infk = jnp.full((L,), INF, jnp.float32)
    zv = jnp.zeros((L,), jnp.int32)
    two = []
    for i in range(0, len(blocks) - 1, 2):
        two.append(_merge16_16(blocks[i], blocks[i + 1]))
    if len(blocks) % 2:
        two.append([blocks[-1], (infk, zv)])
    while len(two) > 1:
        nxt = []
        for i in range(0, len(two) - 1, 2):
            nxt.append(_merge32_low(two[i], two[i + 1]))
        if len(two) % 2:
            nxt.append(two[-1])
        two = nxt
    return two[0]


def _sc_body(d2rows, ids_hbm, vel_hbm, out_hbm,
             idsvm, cand, velbuf, ubuf, sem):
    from jax.experimental.pallas import tpu_sc as plsc
    wid = lax.axis_index("c") * 16 + lax.axis_index("s")
    base = wid * ROWS
    pltpu.sync_copy(ids_hbm.at[pl.ds(base, ROWS)], idsvm)
    iota = lax.iota(jnp.int32, L)

    def row_body(r, carry):
        row = base + r
        bidA = idsvm[r, pl.ds(0, L)]
        bidB = idsvm[r, pl.ds(L, L)]
        segA = bidA * B + row
        segB = bidB * B + row
        cpA = pltpu.async_copy(d2rows.at[segA], cand.at[pl.ds(0, L)], sem)
        cpB = pltpu.async_copy(d2rows.at[segB], cand.at[pl.ds(L, L)], sem)
        cpA.wait()
        cpB.wait()

        cls_blocks = []
        for c in range(BUCKET // L):
            m = cand[0, pl.ds(c * L, L)]
            for j in range(1, NSEL):
                m = jnp.minimum(m, cand[j, pl.ds(c * L, L)])
            cls_blocks.append(_sort1((m, iota + c * L)))
        Fc = _tournament(cls_blocks)

        blocks = []
        for i in range(K):
            p = Fc[0][1][i] if i < L else Fc[1][1][i - L]
            vA = plsc.load_gather(cand, [iota, jnp.broadcast_to(p, (L,))])
            vB = plsc.load_gather(cand, [iota + L, jnp.broadcast_to(p, (L,))])
            blocks.append(_sort1((vA, bidA * BUCKET + p)))
            blocks.append(_sort1((vB, bidB * BUCKET + p)))
        F = _tournament(blocks)

        h2 = jnp.maximum(F[1][0][K - L - 1], 1e-12)
        h2v = jnp.broadcast_to(h2, (L,))
        wA = jnp.exp(-jnp.maximum(F[0][0], 1e-12) / (2.0 * h2v))
        mB = iota < (K - L)
        wB = jnp.where(mB, jnp.exp(-jnp.maximum(F[1][0], 1e-12) / (2.0 * h2v)),
                       0.0)
        s = jnp.sum(wA) + jnp.sum(wB) + 1e-12
        wA = wA / s
        wB = wB / s

        colA = F[0][1]
        colB = jnp.where(mB, F[1][1], 0)
        gA = pltpu.async_copy(vel_hbm.at[colA], velbuf.at[pl.ds(0, L)], sem)
        gB = pltpu.async_copy(vel_hbm.at[colB], velbuf.at[pl.ds(L, L)], sem)
        gA.wait()
        gB.wait()
        for c in range(D // L):
            acc = jnp.zeros((L,), jnp.float32)
            for j in range(K):
                wj = wA[j] if j < L else wB[j - L]
                acc = acc + wj * velbuf[j, pl.ds(c * L, L)]
            ubuf[r, pl.ds(c * L, L)] = acc
        return carry

    lax.fori_loop(0, ROWS, row_body, 0)
    pltpu.sync_copy(ubuf, out_hbm.at[pl.ds(base, ROWS)])


def _sc_topk_combine(d2rows, ids, velocity_p):
    from jax.experimental.pallas import tpu_sc as plsc
    mesh = plsc.VectorSubcoreMesh(core_axis_name="c", subcore_axis_name="s",
                                  num_cores=2, num_subcores=16)
    f = functools.partial(
        pl.kernel,
        out_type=jax.ShapeDtypeStruct((B, BUCKET), jnp.float32),
        mesh=mesh,
        compiler_params=pltpu.CompilerParams(needs_layout_passes=False,
                                             use_tc_tiling_on_sc=False),
        scratch_types=[
            pltpu.VMEM((ROWS, BUCKET), jnp.int32),
            pltpu.VMEM((NSEL, BUCKET), jnp.float32),
            pltpu.VMEM((32, BUCKET), jnp.float32),
            pltpu.VMEM((ROWS, BUCKET), jnp.float32),
            pltpu.SemaphoreType.DMA,
        ],
    )(_sc_body)
    return f(d2rows, ids, velocity_p)


@jax.jit
def kernel(t, z, data, velocity, W, b):
    x = z[:, :D]
    d2, mins = pl.pallas_call(
        _d2_kernel,
        grid=(NCHUNK,),
        in_specs=[
            pl.BlockSpec((B, D), lambda j: (0, 0)),
            pl.BlockSpec((CHUNK, D), lambda j: (j, 0)),
        ],
        out_specs=[
            pl.BlockSpec((BPB * B, BUCKET), lambda j: (j, 0)),
            pl.BlockSpec((1, B, BPB), lambda j: (j, 0, 0)),
        ],
        out_shape=[
            jax.ShapeDtypeStruct((NBUCKETS * B, BUCKET), jnp.float32),
            jax.ShapeDtypeStruct((NCHUNK, B, BPB), jnp.float32),
        ],
    )(x, data)
    mins = mins.transpose(1, 0, 2).reshape(B, NBUCKETS)

    ids = pl.pallas_call(
        _select_kernel,
        grid=(B // BT,),
        in_specs=[pl.BlockSpec((BT, NBUCKETS), lambda i: (i, 0))],
        out_specs=pl.BlockSpec((BT, 128), lambda i: (i, 0)),
        out_shape=jax.ShapeDtypeStruct((B, 128), jnp.int32),
    )(mins)

    velocity_p = jnp.pad(velocity, ((0, 0), (0, BUCKET - D)))
    u = _sc_topk_combine(d2, ids, velocity_p)

    out = pl.pallas_call(
        _combine_kernel,
        grid=(B // BT,),
        in_specs=[
            pl.BlockSpec((BT, D + 2), lambda i: (i, 0)),
            pl.BlockSpec((D + 1, D), lambda i: (0, 0)),
            pl.BlockSpec((1, D), lambda i: (0, 0)),
            pl.BlockSpec((1, 1), lambda i: (0, 0)),
            pl.BlockSpec((BT, BUCKET), lambda i: (i, 0)),
        ],
        out_specs=pl.BlockSpec((BT, D + 2), lambda i: (i, 0)),
        out_shape=jax.ShapeDtypeStruct((B, D + 2), jnp.float32),
    )(z, W, b.reshape(1, D), t.reshape(1, 1), u)
    return out


# speedup vs baseline: 1.1108x; 1.1108x over previous
import functools
import jax
import jax.numpy as jnp
from jax import lax
from jax.experimental import pallas as pl
from jax.experimental.pallas import tpu as pltpu

B = 1024
N = 100000
D = 64
K = 20
CHUNK = 2048
BUCKET = 128
NCHUNK = (N + CHUNK - 1) // CHUNK
NPAD = NCHUNK * CHUNK
NBUCKETS = NPAD // BUCKET
BPB = CHUNK // BUCKET
BT = 128
INF = float("inf")


def _d2_kernel(x_ref, data_ref, d2_ref, mins_ref):
    j = pl.program_id(0)
    x = x_ref[...]
    dat = data_ref[...]
    x2 = jnp.sum(x * x, axis=1, keepdims=True)
    y2 = jnp.sum(dat * dat, axis=1)[None, :]
    xy = jax.lax.dot_general(x, dat, (((1,), (1,)), ((), ())),
                             preferred_element_type=jnp.float32)
    d2 = x2 + y2 - 2.0 * xy
    cols = j * CHUNK + jax.lax.broadcasted_iota(jnp.int32, (1, CHUNK), 1)
    d2 = jnp.where(cols < N, d2, INF)
    mins = []
    for bsub in range(BPB):
        d2b = d2[:, bsub * BUCKET:(bsub + 1) * BUCKET]
        d2_ref[pl.ds(bsub * B, B), :] = d2b
        mins.append(jnp.min(d2b, axis=-1, keepdims=True))
    mins_ref[...] = jnp.concatenate(mins, axis=1)[None]


def _select_kernel(mins_ref, ids_ref):
    m = mins_ref[...]
    col = jax.lax.broadcasted_iota(jnp.int32, (BT, NBUCKETS), 1)
    ids = []
    for _ in range(K):
        cur = jnp.min(m, axis=1, keepdims=True)
        idx = jnp.min(jnp.where(m == cur, col, jnp.int32(2 ** 30)),
                      axis=1, keepdims=True)
        ids.append(idx)
        m = jnp.where(col == idx, INF, m)
    pad = jnp.full((BT, 1), NBUCKETS - 1, jnp.int32)
    ids.extend([pad] * (128 - K))
    ids_ref[...] = jnp.concatenate(ids, axis=1)


def _combine_kernel(z_ref, w_ref, b_ref, t_ref, u_ref, out_ref):
    z = z_ref[...]
    x = z[:, :D]
    w = w_ref[...]
    t = t_ref[0, 0]
    u = u_ref[...][:, :D]
    xdot = jax.lax.dot_general(x, w[:D, :], (((1,), (0,)), ((), ())),
                               preferred_element_type=jnp.float32)
    xdot = xdot + t * w[D:D + 1, :] + b_ref[...]
    num = jnp.sum(u * xdot, axis=1, keepdims=True)
    nu = jnp.sqrt(jnp.sum(u * u, axis=1, keepdims=True))
    nx = jnp.sqrt(jnp.sum(xdot * xdot, axis=1, keepdims=True))
    den = jnp.maximum(nu, 1e-8) * jnp.maximum(nx, 1e-8)
    cos = 1.0 - num / den
    l2 = jnp.sum((u - xdot) ** 2, axis=1, keepdims=True)
    out_ref[...] = jnp.concatenate([xdot, cos, l2], axis=1)


NW = 32
ROWS = B // NW
NSEL = 32
L = 16


def _sel(p, a, b):
    return (jnp.where(p, a[0], b[0]), jnp.where(p, a[1], b[1]))


def _rev(a):
    return (lax.rev(a[0], (0,)), lax.rev(a[1], (0,)))


def _sort1(a):
    from jax.experimental.pallas import tpu_sc as plsc
    k, v = plsc.sort_key_val(a[0], a[1])
    return (k, v)


def _merge16_16(a, b):
    rb = _rev(b)
    p = a[0] <= rb[0]
    lo = _sel(p, a, rb)
    hi = _sel(p, rb, a)
    return [_sort1(lo), _sort1(hi)]


def _merge32_low(A, Bb):
    rb1 = _rev(Bb[1])
    rb0 = _rev(Bb[0])
    p0 = A[0][0] <= rb1[0]
    l0 = _sel(p0, A[0], rb1)
    p1 = A[1][0] <= rb0[0]
    l1 = _sel(p1, A[1], rb0)
    q = l0[0] <= l1[0]
    m0 = _sel(q, l0, l1)
    m1 = _sel(q, l1, l0)
    return [_sort1(m0), _sort1(m1)]


def _tournament(blocks):
    infk = jnp.full((L,), INF, jnp.float32)
    zv = jnp.zeros((L,), jnp.int32)
    two = []
    for i in range(0, len(blocks) - 1, 2):
        two.append(_merge16_16(blocks[i], blocks[i + 1]))
    if len(blocks) % 2:
        two.append([blocks[-1], (infk, zv)])
    while len(two) > 1:
        nxt = []
        for i in range(0, len(two) - 1, 2):
            nxt.append(_merge32_low(two[i], two[i + 1]))
        if len(two) % 2:
            nxt.append(two[-1])
        two = nxt
    return two[0]


def _sc_body(d2rows, ids_hbm, vel_hbm, out_hbm,
             idsvm, candA, candB, velbuf, ubuf, semA, semB, semV):
    from jax.experimental.pallas import tpu_sc as plsc
    wid = lax.axis_index("c") * 16 + lax.axis_index("s")
    base = wid * ROWS
    pltpu.sync_copy(ids_hbm.at[pl.ds(base, ROWS)], idsvm)
    iota = lax.iota(jnp.int32, L)

    def issue(rr, cnd, sem):
        rc = jnp.minimum(rr, ROWS - 1)
        bA = idsvm[rc, pl.ds(0, L)]
        bB = idsvm[rc, pl.ds(L, L)]
        row = base + rc
        pltpu.async_copy(d2rows.at[bA * B + row], cnd.at[pl.ds(0, L)], sem)
        pltpu.async_copy(d2rows.at[bB * B + row], cnd.at[pl.ds(L, L)], sem)

    def drain(cnd, sem):
        pltpu.make_async_copy(d2rows.at[iota], cnd.at[pl.ds(0, L)], sem).wait()
        pltpu.make_async_copy(d2rows.at[iota], cnd.at[pl.ds(L, L)], sem).wait()

    def process(r, cand):
        bidA = idsvm[r, pl.ds(0, L)]
        bidB = idsvm[r, pl.ds(L, L)]
        cls_blocks = []
        for c in range(BUCKET // L):
            m = cand[0, pl.ds(c * L, L)]
            for j in range(1, K):
                m = jnp.minimum(m, cand[j, pl.ds(c * L, L)])
            cls_blocks.append(_sort1((m, iota + c * L)))
        Fc = _tournament(cls_blocks)

        blocks = []
        for i in range(K):
            p = Fc[0][1][i] if i < L else Fc[1][1][i - L]
            vA = plsc.load_gather(cand, [iota, jnp.broadcast_to(p, (L,))])
            vB = plsc.load_gather(cand, [iota + L, jnp.broadcast_to(p, (L,))])
            blocks.append(_sort1((vA, bidA * BUCKET + p)))
            blocks.append(_sort1((vB, bidB * BUCKET + p)))
        F = _tournament(blocks)

        h2 = jnp.maximum(F[1][0][K - L - 1], 1e-12)
        h2v = jnp.broadcast_to(h2, (L,))
        wA = jnp.exp(-jnp.maximum(F[0][0], 1e-12) / (2.0 * h2v))
        mB = iota < (K - L)
        wB = jnp.where(mB, jnp.exp(-jnp.maximum(F[1][0], 1e-12) / (2.0 * h2v)),
                       0.0)
        s = jnp.sum(wA) + jnp.sum(wB) + 1e-12
        wA = wA / s
        wB = wB / s

        colA = F[0][1]
        colB = jnp.where(mB, F[1][1], 0)
        gA = pltpu.async_copy(vel_hbm.at[colA], velbuf.at[pl.ds(0, L)], semV)
        gB = pltpu.async_copy(vel_hbm.at[colB], velbuf.at[pl.ds(L, L)], semV)
        gA.wait()
        gB.wait()
        for c in range(D // L):
            acc = jnp.zeros((L,), jnp.float32)
            for j in range(K):
                wj = wA[j] if j < L else wB[j - L]
                acc = acc + wj * velbuf[j, pl.ds(c * L, L)]
            ubuf[r, pl.ds(c * L, L)] = acc

    issue(jnp.int32(0), candA, semA)
    issue(jnp.int32(1), candB, semB)

    def pair_body(g, carry):
        drain(candA, semA)
        process(2 * g, candA)
        issue(2 * g + 2, candA, semA)
        drain(candB, semB)
        process(2 * g + 1, candB)
        issue(2 * g + 3, candB, semB)
        return carry

    lax.fori_loop(0, ROWS // 2, pair_body, 0)
    drain(candA, semA)
    drain(candB, semB)
    pltpu.sync_copy(ubuf, out_hbm.at[pl.ds(base, ROWS)])


def _sc_topk_combine(d2rows, ids, velocity_p):
    from jax.experimental.pallas import tpu_sc as plsc
    mesh = plsc.VectorSubcoreMesh(core_axis_name="c", subcore_axis_name="s",
                                  num_cores=2, num_subcores=16)
    f = functools.partial(
        pl.kernel,
        out_type=jax.ShapeDtypeStruct((B, BUCKET), jnp.float32),
        mesh=mesh,
        compiler_params=pltpu.CompilerParams(needs_layout_passes=False,
                                             use_tc_tiling_on_sc=False),
        scratch_types=[
            pltpu.VMEM((ROWS, BUCKET), jnp.int32),
            pltpu.VMEM((NSEL, BUCKET), jnp.float32),
            pltpu.VMEM((NSEL, BUCKET), jnp.float32),
            pltpu.VMEM((32, BUCKET), jnp.float32),
            pltpu.VMEM((ROWS, BUCKET), jnp.float32),
            pltpu.SemaphoreType.DMA,
            pltpu.SemaphoreType.DMA,
            pltpu.SemaphoreType.DMA,
        ],
    )(_sc_body)
    return f(d2rows, ids, velocity_p)


@jax.jit
def kernel(t, z, data, velocity, W, b):
    x = z[:, :D]
    d2, mins = pl.pallas_call(
        _d2_kernel,
        grid=(NCHUNK,),
        in_specs=[
            pl.BlockSpec((B, D), lambda j: (0, 0)),
            pl.BlockSpec((CHUNK, D), lambda j: (j, 0)),
        ],
        out_specs=[
            pl.BlockSpec((BPB * B, BUCKET), lambda j: (j, 0)),
            pl.BlockSpec((1, B, BPB), lambda j: (j, 0, 0)),
        ],
        out_shape=[
            jax.ShapeDtypeStruct((NBUCKETS * B, BUCKET), jnp.float32),
            jax.ShapeDtypeStruct((NCHUNK, B, BPB), jnp.float32),
        ],
    )(x, data)
    mins = mins.transpose(1, 0, 2).reshape(B, NBUCKETS)

    ids = pl.pallas_call(
        _select_kernel,
        grid=(B // BT,),
        in_specs=[pl.BlockSpec((BT, NBUCKETS), lambda i: (i, 0))],
        out_specs=pl.BlockSpec((BT, 128), lambda i: (i, 0)),
        out_shape=jax.ShapeDtypeStruct((B, 128), jnp.int32),
    )(mins)

    velocity_p = jnp.pad(velocity, ((0, 0), (0, BUCKET - D)))
    u = _sc_topk_combine(d2, ids, velocity_p)

    out = pl.pallas_call(
        _combine_kernel,
        grid=(B // BT,),
        in_specs=[
            pl.BlockSpec((BT, D + 2), lambda i: (i, 0)),
            pl.BlockSpec((D + 1, D), lambda i: (0, 0)),
            pl.BlockSpec((1, D), lambda i: (0, 0)),
            pl.BlockSpec((1, 1), lambda i: (0, 0)),
            pl.BlockSpec((BT, BUCKET), lambda i: (i, 0)),
        ],
        out_specs=pl.BlockSpec((BT, D + 2), lambda i: (i, 0)),
        out_shape=jax.ShapeDtypeStruct((B, D + 2), jnp.float32),
    )(z, W, b.reshape(1, D), t.reshape(1, 1), u)
    return out


# speedup vs baseline: 1.1121x; 1.0011x over previous
import functools
import jax
import jax.numpy as jnp
from jax import lax
from jax.experimental import pallas as pl
from jax.experimental.pallas import tpu as pltpu

B = 1024
N = 100000
D = 64
K = 20
CHUNK = 2048
BUCKET = 128
NCHUNK = (N + CHUNK - 1) // CHUNK
NPAD = NCHUNK * CHUNK
NBUCKETS = NPAD // BUCKET
BPB = CHUNK // BUCKET
BT = 128
INF = float("inf")


def _d2_kernel(x_ref, data_ref, d2_ref, mins_ref):
    j = pl.program_id(0)
    x = x_ref[...]
    dat = data_ref[...]
    x2 = jnp.sum(x * x, axis=1, keepdims=True)
    y2 = jnp.sum(dat * dat, axis=1)[None, :]
    xy = jax.lax.dot_general(x, dat, (((1,), (1,)), ((), ())),
                             preferred_element_type=jnp.float32)
    d2 = x2 + y2 - 2.0 * xy
    cols = j * CHUNK + jax.lax.broadcasted_iota(jnp.int32, (1, CHUNK), 1)
    d2 = jnp.where(cols < N, d2, INF)
    mins = []
    for bsub in range(BPB):
        d2b = d2[:, bsub * BUCKET:(bsub + 1) * BUCKET]
        d2_ref[pl.ds(bsub * B, B), :] = d2b
        mins.append(jnp.min(d2b, axis=-1, keepdims=True))
    mins_ref[...] = jnp.concatenate(mins, axis=1)[None]


def _select_kernel(mins_ref, ids_ref):
    m = mins_ref[...]
    col = jax.lax.broadcasted_iota(jnp.int32, (BT, NBUCKETS), 1)
    ids = []
    for _ in range(K):
        cur = jnp.min(m, axis=1, keepdims=True)
        idx = jnp.min(jnp.where(m == cur, col, jnp.int32(2 ** 30)),
                      axis=1, keepdims=True)
        ids.append(idx)
        m = jnp.where(col == idx, INF, m)
    pad = jnp.full((BT, 1), NBUCKETS - 1, jnp.int32)
    ids.extend([pad] * (128 - K))
    ids_ref[...] = jnp.concatenate(ids, axis=1)


def _combine_kernel(z_ref, w_ref, b_ref, t_ref, u_ref, out_ref):
    z = z_ref[...]
    x = z[:, :D]
    w = w_ref[...]
    t = t_ref[0, 0]
    u = u_ref[...][:, :D]
    xdot = jax.lax.dot_general(x, w[:D, :], (((1,), (0,)), ((), ())),
                               preferred_element_type=jnp.float32)
    xdot = xdot + t * w[D:D + 1, :] + b_ref[...]
    num = jnp.sum(u * xdot, axis=1, keepdims=True)
    nu = jnp.sqrt(jnp.sum(u * u, axis=1, keepdims=True))
    nx = jnp.sqrt(jnp.sum(xdot * xdot, axis=1, keepdims=True))
    den = jnp.maximum(nu, 1e-8) * jnp.maximum(nx, 1e-8)
    cos = 1.0 - num / den
    l2 = jnp.sum((u - xdot) ** 2, axis=1, keepdims=True)
    out_ref[...] = jnp.concatenate([xdot, cos, l2], axis=1)


NW = 32
ROWS = B // NW
NSEL = 32
L = 16


def _sel(p, a, b):
    return (jnp.where(p, a[0], b[0]), jnp.where(p, a[1], b[1]))


def _rev(a):
    return (lax.rev(a[0], (0,)), lax.rev(a[1], (0,)))


def _sort1(a):
    from jax.experimental.pallas import tpu_sc as plsc
    k, v = plsc.sort_key_val(a[0], a[1])
    return (k, v)


def _merge16_16(a, b):
    rb = _rev(b)
    p = a[0] <= rb[0]
    lo = _sel(p, a, rb)
    hi = _sel(p, rb, a)
    return [_sort1(lo), _sort1(hi)]


def _merge32_low(A, Bb):
    rb1 = _rev(Bb[1])
    rb0 = _rev(Bb[0])
    p0 = A[0][0] <= rb1[0]
    l0 = _sel(p0, A[0], rb1)
    p1 = A[1][0] <= rb0[0]
    l1 = _sel(p1, A[1], rb0)
    q = l0[0] <= l1[0]
    m0 = _sel(q, l0, l1)
    m1 = _sel(q, l1, l0)
    return [_sort1(m0), _sort1(m1)]


def _tournament(blocks):
    infk = jnp.full((L,), INF, jnp.float32)
    zv = jnp.zeros((L,), jnp.int32)
    two = []
    for i in range(0, len(blocks) - 1, 2):
        two.append(_merge16_16(blocks[i], blocks[i + 1]))
    if len(blocks) % 2:
        two.append([blocks[-1], (infk, zv)])
    while len(two) > 1:
        nxt = []
        for i in range(0, len(two) - 1, 2):
            nxt.append(_merge32_low(two[i], two[i + 1]))
        if len(two) % 2:
            nxt.append(two[-1])
        two = nxt
    return two[0]


def _sc_body(d2rows, ids_hbm, vel_hbm, out_hbm,
             idsvm, candA, candB, velbufA, velbufB, ubuf,
             semA, semB, semVA, semVB):
    from jax.experimental.pallas import tpu_sc as plsc
    wid = lax.axis_index("c") * 16 + lax.axis_index("s")
    base = wid * ROWS
    pltpu.sync_copy(ids_hbm.at[pl.ds(base, ROWS)], idsvm)
    iota = lax.iota(jnp.int32, L)

    def issue(rr, cnd, sem):
        rc = jnp.minimum(rr, ROWS - 1)
        bA = idsvm[rc, pl.ds(0, L)]
        bB = idsvm[rc, pl.ds(L, L)]
        row = base + rc
        pltpu.async_copy(d2rows.at[bA * B + row], cnd.at[pl.ds(0, L)], sem)
        pltpu.async_copy(d2rows.at[bB * B + row], cnd.at[pl.ds(L, L)], sem)

    def drain(cnd, sem):
        pltpu.make_async_copy(d2rows.at[iota], cnd.at[pl.ds(0, L)], sem).wait()
        pltpu.make_async_copy(d2rows.at[iota], cnd.at[pl.ds(L, L)], sem).wait()

    def select(r, cand, velbuf, semV):
        bidA = idsvm[r, pl.ds(0, L)]
        bidB = idsvm[r, pl.ds(L, L)]
        cls_blocks = []
        for c in range(BUCKET // L):
            m = cand[0, pl.ds(c * L, L)]
            for j in range(1, K):
                m = jnp.minimum(m, cand[j, pl.ds(c * L, L)])
            cls_blocks.append(_sort1((m, iota + c * L)))
        Fc = _tournament(cls_blocks)

        blocks = []
        for i in range(K):
            p = Fc[0][1][i] if i < L else Fc[1][1][i - L]
            vA = plsc.load_gather(cand, [iota, jnp.broadcast_to(p, (L,))])
            vB = plsc.load_gather(cand, [iota + L, jnp.broadcast_to(p, (L,))])
            blocks.append(_sort1((vA, bidA * BUCKET + p)))
            blocks.append(_sort1((vB, bidB * BUCKET + p)))
        F = _tournament(blocks)

        h2 = jnp.maximum(F[1][0][K - L - 1], 1e-12)
        h2v = jnp.broadcast_to(h2, (L,))
        wA = jnp.exp(-jnp.maximum(F[0][0], 1e-12) / (2.0 * h2v))
        mB = iota < (K - L)
        wB = jnp.where(mB, jnp.exp(-jnp.maximum(F[1][0], 1e-12) / (2.0 * h2v)),
                       0.0)
        s = jnp.sum(wA) + jnp.sum(wB) + 1e-12

        colA = F[0][1]
        colB = jnp.where(mB, F[1][1], 0)
        pltpu.async_copy(vel_hbm.at[colA], velbuf.at[pl.ds(0, L)], semV)
        pltpu.async_copy(vel_hbm.at[colB], velbuf.at[pl.ds(L, L)], semV)
        return wA / s, wB / s

    def combine(r, w, velbuf, semV):
        wA, wB = w
        pltpu.make_async_copy(vel_hbm.at[iota],
                              velbuf.at[pl.ds(0, L)], semV).wait()
        pltpu.make_async_copy(vel_hbm.at[iota],
                              velbuf.at[pl.ds(L, L)], semV).wait()
        for c in range(D // L):
            acc = jnp.zeros((L,), jnp.float32)
            for j in range(K):
                wj = wA[j] if j < L else wB[j - L]
                acc = acc + wj * velbuf[j, pl.ds(c * L, L)]
            ubuf[r, pl.ds(c * L, L)] = acc

    issue(jnp.int32(0), candA, semA)
    issue(jnp.int32(1), candB, semB)

    def pair_body(g, carry):
        drain(candA, semA)
        wa = select(2 * g, candA, velbufA, semVA)
        issue(2 * g + 2, candA, semA)
        drain(candB, semB)
        wb = select(2 * g + 1, candB, velbufB, semVB)
        issue(2 * g + 3, candB, semB)
        combine(2 * g, wa, velbufA, semVA)
        combine(2 * g + 1, wb, velbufB, semVB)
        return carry

    lax.fori_loop(0, ROWS // 2, pair_body, 0)
    drain(candA, semA)
    drain(candB, semB)
    pltpu.sync_copy(ubuf, out_hbm.at[pl.ds(base, ROWS)])


def _sc_topk_combine(d2rows, ids, velocity_p):
    from jax.experimental.pallas import tpu_sc as plsc
    mesh = plsc.VectorSubcoreMesh(core_axis_name="c", subcore_axis_name="s",
                                  num_cores=2, num_subcores=16)
    f = functools.partial(
        pl.kernel,
        out_type=jax.ShapeDtypeStruct((B, BUCKET), jnp.float32),
        mesh=mesh,
        compiler_params=pltpu.CompilerParams(needs_layout_passes=False,
                                             use_tc_tiling_on_sc=False),
        scratch_types=[
            pltpu.VMEM((ROWS, BUCKET), jnp.int32),
            pltpu.VMEM((NSEL, BUCKET), jnp.float32),
            pltpu.VMEM((NSEL, BUCKET), jnp.float32),
            pltpu.VMEM((32, BUCKET), jnp.float32),
            pltpu.VMEM((32, BUCKET), jnp.float32),
            pltpu.VMEM((ROWS, BUCKET), jnp.float32),
            pltpu.SemaphoreType.DMA,
            pltpu.SemaphoreType.DMA,
            pltpu.SemaphoreType.DMA,
            pltpu.SemaphoreType.DMA,
        ],
    )(_sc_body)
    return f(d2rows, ids, velocity_p)


@jax.jit
def kernel(t, z, data, velocity, W, b):
    x = z[:, :D]
    d2, mins = pl.pallas_call(
        _d2_kernel,
        grid=(NCHUNK,),
        in_specs=[
            pl.BlockSpec((B, D), lambda j: (0, 0)),
            pl.BlockSpec((CHUNK, D), lambda j: (j, 0)),
        ],
        out_specs=[
            pl.BlockSpec((BPB * B, BUCKET), lambda j: (j, 0)),
            pl.BlockSpec((1, B, BPB), lambda j: (j, 0, 0)),
        ],
        out_shape=[
            jax.ShapeDtypeStruct((NBUCKETS * B, BUCKET), jnp.float32),
            jax.ShapeDtypeStruct((NCHUNK, B, BPB), jnp.float32),
        ],
    )(x, data)
    mins = mins.transpose(1, 0, 2).reshape(B, NBUCKETS)

    ids = pl.pallas_call(
        _select_kernel,
        grid=(B // BT,),
        in_specs=[pl.BlockSpec((BT, NBUCKETS), lambda i: (i, 0))],
        out_specs=pl.BlockSpec((BT, 128), lambda i: (i, 0)),
        out_shape=jax.ShapeDtypeStruct((B, 128), jnp.int32),
    )(mins)

    velocity_p = jnp.pad(velocity, ((0, 0), (0, BUCKET - D)))
    u = _sc_topk_combine(d2, ids, velocity_p)

    out = pl.pallas_call(
        _combine_kernel,
        grid=(B // BT,),
        in_specs=[
            pl.BlockSpec((BT, D + 2), lambda i: (i, 0)),
            pl.BlockSpec((D + 1, D), lambda i: (0, 0)),
            pl.BlockSpec((1, D), lambda i: (0, 0)),
            pl.BlockSpec((1, 1), lambda i: (0, 0)),
            pl.BlockSpec((BT, BUCKET), lambda i: (i, 0)),
        ],
        out_specs=pl.BlockSpec((BT, D + 2), lambda i: (i, 0)),
        out_shape=jax.ShapeDtypeStruct((B, D + 2), jnp.float32),
    )(z, W, b.reshape(1, D), t.reshape(1, 1), u)
    return out
